# baseline (speedup 1.0000x reference)
import functools

import jax
import jax.numpy as jnp
from jax import lax
from jax.experimental import pallas as pl
from jax.experimental.pallas import tpu as pltpu
from jax.experimental.pallas import tpu_sc as plsc

N = 10000
E = 320000
IN_CH = 128
D = 64
K = 3
ALPHA = 0.1

NC = 2
NS = 16
NW = NC * NS
L = 16

N_PAD = 10240
RPT = N_PAD // NS
B = 128
CH = 80
CHX = CH + 2
E_TILE = B * CH
E_PAD = E_TILE * NW

ROWBLK = 1024


def _mesh():
    return plsc.VectorSubcoreMesh(
        core_axis_name="c", subcore_axis_name="s",
        num_cores=NC, num_subcores=NS)


_SC_PARAMS = pltpu.CompilerParams(use_tc_tiling_on_sc=False)


@functools.partial(
    pl.kernel,
    out_type=jax.ShapeDtypeStruct((NC, 2, N_PAD), jnp.float32),
    mesh=_mesh(),
    compiler_params=_SC_PARAMS,
    scratch_types=[
        pltpu.VMEM((CHX, B), jnp.int32),
        pltpu.VMEM((CHX, B), jnp.int32),
        pltpu.VMEM((B,), jnp.float32),
        pltpu.VMEM((RPT,), jnp.float32),
        pltpu.VMEM_SHARED((N_PAD,), jnp.float32),
        pltpu.VMEM_SHARED((N_PAD,), jnp.float32),
    ],
)
def _deg_kernel(src_h, dst_h, out_h, src_v, dst_v, ones_v, zero_v,
                dego_sh, degi_sh):
    cid = lax.axis_index("c")
    sid = lax.axis_index("s")
    wid = cid * NS + sid

    def zi(i, c):
        zero_v[pl.ds(i * L, L)] = jnp.zeros((L,), jnp.float32)
        return c
    lax.fori_loop(0, RPT // L, zi, 0)

    def oi(i, c):
        ones_v[pl.ds(i * L, L)] = jnp.ones((L,), jnp.float32)
        return c
    lax.fori_loop(0, B // L, oi, 0)

    pltpu.sync_copy(zero_v, dego_sh.at[pl.ds(sid * RPT, RPT)])
    pltpu.sync_copy(zero_v, degi_sh.at[pl.ds(sid * RPT, RPT)])
    pltpu.sync_copy(src_h.at[wid], src_v)
    pltpu.sync_copy(dst_h.at[wid], dst_v)
    plsc.subcore_barrier()

    def body(j, c):
        pltpu.sync_copy(ones_v, dego_sh.at[src_v.at[j]], add=True)
        pltpu.sync_copy(ones_v, degi_sh.at[dst_v.at[j]], add=True)
        return c
    lax.fori_loop(0, CH, body, 0)

    plsc.subcore_barrier()
    sl = pl.ds(sid * RPT, RPT)
    pltpu.sync_copy(dego_sh.at[sl], out_h.at[cid, 0, sl])
    pltpu.sync_copy(degi_sh.at[sl], out_h.at[cid, 1, sl])


@functools.partial(
    pl.kernel,
    out_type=jax.ShapeDtypeStruct((NW, CH, B), jnp.float32),
    mesh=_mesh(),
    compiler_params=_SC_PARAMS,
    scratch_types=[
        pltpu.VMEM((CHX, B), jnp.int32),
        pltpu.VMEM((CHX, B), jnp.int32),
        pltpu.VMEM((CH, B), jnp.float32),
        pltpu.VMEM((CH, B), jnp.float32),
        pltpu.VMEM((B,), jnp.float32),
        pltpu.VMEM((B,), jnp.float32),
        pltpu.VMEM((B,), jnp.float32),
        pltpu.VMEM((B,), jnp.float32),
        pltpu.SemaphoreType.DMA,
        pltpu.SemaphoreType.DMA,
    ],
)
def _coef_kernel(src_h, dst_h, ew_h, no_h, ni_h, out_h,
                 src_v, dst_v, ew_v, coef_v,
                 nog0, nig0, nog1, nig1, sem0, sem1):
    cid = lax.axis_index("c")
    sid = lax.axis_index("s")
    wid = cid * NS + sid
    pltpu.sync_copy(src_h.at[wid], src_v)
    pltpu.sync_copy(dst_h.at[wid], dst_v)
    pltpu.sync_copy(ew_h.at[wid], ew_v)

    pltpu.async_copy(no_h.at[src_v.at[0]], nog0, sem0)
    pltpu.async_copy(ni_h.at[dst_v.at[0]], nig0, sem0)
    pltpu.async_copy(no_h.at[src_v.at[1]], nog1, sem1)
    pltpu.async_copy(ni_h.at[dst_v.at[1]], nig1, sem1)

    def half(t, nog, nig, sem):
        pltpu.make_async_copy(no_h.at[src_v.at[t]], nog, sem).wait()
        pltpu.make_async_copy(ni_h.at[dst_v.at[t]], nig, sem).wait()
        for g in range(B // L):
            sl = pl.ds(g * L, L)
            coef_v[t, sl] = nog[sl] * ew_v[t, sl] * nig[sl]
        pltpu.async_copy(no_h.at[src_v.at[t + 2]], nog, sem)
        pltpu.async_copy(ni_h.at[dst_v.at[t + 2]], nig, sem)

    def body(i, c):
        t = 2 * i
        half(t, nog0, nig0, sem0)
        half(t + 1, nog1, nig1, sem1)
        return c
    lax.fori_loop(0, CH // 2, body, 0)

    pltpu.make_async_copy(no_h.at[src_v.at[0]], nog0, sem0).wait()
    pltpu.make_async_copy(ni_h.at[dst_v.at[0]], nig0, sem0).wait()
    pltpu.make_async_copy(no_h.at[src_v.at[1]], nog1, sem1).wait()
    pltpu.make_async_copy(ni_h.at[dst_v.at[1]], nig1, sem1).wait()
    pltpu.sync_copy(coef_v, out_h.at[wid])


@functools.partial(
    pl.kernel,
    out_type=jax.ShapeDtypeStruct((NC, N_PAD, D), jnp.float32),
    mesh=_mesh(),
    compiler_params=_SC_PARAMS,
    scratch_types=[
        pltpu.VMEM((CHX, B), jnp.int32),
        pltpu.VMEM((CHX, B), jnp.int32),
        pltpu.VMEM((CH, B), jnp.float32),
        pltpu.VMEM((B, D), jnp.float32),
        pltpu.VMEM((B, D), jnp.float32),
        pltpu.VMEM((B, D), jnp.float32),
        pltpu.VMEM_SHARED((N_PAD, D), jnp.float32),
        pltpu.SemaphoreType.DMA,
        pltpu.SemaphoreType.DMA,
    ],
)
def _prop_kernel(h_h, src_h, dst_h, coef_h, out_h,
                 src_v, dst_v, coef_v, rows0, rows1, zero_v, agg_sh,
                 sem0, sem1):
    cid = lax.axis_index("c")
    sid = lax.axis_index("s")
    wid = cid * NS + sid

    def zi(i, c):
        zero_v[i // (D // L), pl.ds((i % (D // L)) * L, L)] = (
            jnp.zeros((L,), jnp.float32))
        return c
    lax.fori_loop(0, B * D // L, zi, 0)

    def zc(i, c):
        pltpu.sync_copy(zero_v, agg_sh.at[pl.ds(sid * RPT + i * B, B)])
        return c
    lax.fori_loop(0, RPT // B, zc, 0)

    pltpu.sync_copy(src_h.at[wid], src_v)
    pltpu.sync_copy(dst_h.at[wid], dst_v)
    pltpu.sync_copy(coef_h.at[wid], coef_v)
    plsc.subcore_barrier()

    pltpu.async_copy(h_h.at[src_v.at[0]], rows0, sem0)
    pltpu.async_copy(h_h.at[src_v.at[1]], rows1, sem1)

    def half(t, rows, sem):
        pltpu.make_async_copy(h_h.at[src_v.at[t]], rows, sem).wait()

        def scale(g, cc):
            cvec = coef_v[t, pl.ds(g * L, L)]
            for m in range(L):
                e = g * L + m
                s = cvec[m]
                for k in range(D // L):
                    rows[e, pl.ds(k * L, L)] = rows[e, pl.ds(k * L, L)] * s
            return cc
        lax.fori_loop(0, B // L, scale, 0)
        pltpu.sync_copy(rows, agg_sh.at[dst_v.at[t]], add=True)
        pltpu.async_copy(h_h.at[src_v.at[t + 2]], rows, sem)

    def body(i, c):
        t = 2 * i
        half(t, rows0, sem0)
        half(t + 1, rows1, sem1)
        return c
    lax.fori_loop(0, CH // 2, body, 0)

    pltpu.make_async_copy(h_h.at[src_v.at[0]], rows0, sem0).wait()
    pltpu.make_async_copy(h_h.at[src_v.at[1]], rows1, sem1).wait()

    plsc.subcore_barrier()
    sl = pl.ds(sid * RPT, RPT)
    pltpu.sync_copy(agg_sh.at[sl], out_h.at[cid, sl])


def _mlp_in_call(x_p, w, b2):
    def body(x_r, w_r, b_r, o_r):
        o_r[...] = jnp.maximum(x_r[...] @ w_r[...] + b_r[...], 0.0)
    return pl.pallas_call(
        body,
        grid=(N_PAD // ROWBLK,),
        in_specs=[
            pl.BlockSpec((ROWBLK, IN_CH), lambda i: (i, 0)),
            pl.BlockSpec((IN_CH, D), lambda i: (0, 0)),
            pl.BlockSpec((1, D), lambda i: (0, 0)),
        ],
        out_specs=pl.BlockSpec((ROWBLK, D), lambda i: (i, 0)),
        out_shape=jax.ShapeDtypeStruct((N_PAD, D), jnp.float32),
    )(x_p, w, b2)


_DEG_ROWS = 2 * 2 * N_PAD // 128
_NR = N_PAD // 128


def _norm_call(deg_flat):
    def body(d_r, o_r):
        d = d_r[...]
        tot_o = d[0:_NR] + d[2 * _NR:3 * _NR]
        tot_i = d[_NR:2 * _NR] + d[3 * _NR:4 * _NR]
        no = jnp.where(tot_o > 0, lax.rsqrt(tot_o), 0.0)
        ni = jnp.where(tot_i > 0, lax.rsqrt(tot_i), 0.0)
        o_r[...] = jnp.concatenate([no, ni], axis=0)
    return pl.pallas_call(
        body,
        out_shape=jax.ShapeDtypeStruct((2 * _NR, 128), jnp.float32),
    )(deg_flat)


def _combine_call(part, feat0):
    def body(p_r, f_r, o_r):
        p = p_r[...]
        o_r[...] = (1.0 - ALPHA) * (p[0] + p[1]) + ALPHA * f_r[...]
    return pl.pallas_call(
        body,
        grid=(N_PAD // ROWBLK,),
        in_specs=[
            pl.BlockSpec((NC, ROWBLK, D), lambda i: (0, i, 0)),
            pl.BlockSpec((ROWBLK, D), lambda i: (i, 0)),
        ],
        out_specs=pl.BlockSpec((ROWBLK, D), lambda i: (i, 0)),
        out_shape=jax.ShapeDtypeStruct((N_PAD, D), jnp.float32),
    )(part, feat0)


def _final_call(part, feat0, w, b2):
    def body(p_r, f_r, w_r, b_r, o_r):
        p = p_r[...]
        h = (1.0 - ALPHA) * (p[0] + p[1]) + ALPHA * f_r[...]
        o_r[...] = h @ w_r[...] + b_r[...]
    return pl.pallas_call(
        body,
        grid=(N_PAD // ROWBLK,),
        in_specs=[
            pl.BlockSpec((NC, ROWBLK, D), lambda i: (0, i, 0)),
            pl.BlockSpec((ROWBLK, D), lambda i: (i, 0)),
            pl.BlockSpec((D, D), lambda i: (0, 0)),
            pl.BlockSpec((1, D), lambda i: (0, 0)),
        ],
        out_specs=pl.BlockSpec((ROWBLK, D), lambda i: (i, 0)),
        out_shape=jax.ShapeDtypeStruct((N_PAD, D), jnp.float32),
    )(part, feat0, w, b2)


def _extend(a, fill):
    a_p = jnp.concatenate(
        [a, jnp.full((E_PAD - E,), fill, a.dtype)]).reshape(NW, CH, B)
    return jnp.concatenate(
        [a_p, jnp.full((NW, 2, B), fill, a.dtype)], axis=1)


def kernel(x, edge_index, edge_weight, W_in, b_in, W_out, b_out):
    src_x = _extend(edge_index[0], N)
    dst_x = _extend(edge_index[1], N)
    ew_p = jnp.concatenate(
        [edge_weight, jnp.zeros((E_PAD - E,), jnp.float32)]).reshape(NW, CH, B)
    x_p = jnp.pad(x, ((0, N_PAD - N), (0, 0)))

    deg = _deg_kernel(src_x, dst_x)
    norms = _norm_call(deg.reshape(_DEG_ROWS, 128))
    no = norms[:_NR].reshape(N_PAD)
    ni = norms[_NR:].reshape(N_PAD)

    h0 = _mlp_in_call(x_p, W_in, b_in.reshape(1, D))
    coef = _coef_kernel(src_x, dst_x, ew_p, no, ni)

    h = h0
    out = None
    for t in range(K):
        part = _prop_kernel(h, src_x, dst_x, coef)
        if t < K - 1:
            h = _combine_call(part, h0)
        else:
            out = _final_call(part, h0, W_out, b_out.reshape(1, D))
    return out[:N]


# speedup vs baseline: 1.1430x; 1.1430x over previous
import functools

import jax
import jax.numpy as jnp
from jax import lax
from jax.experimental import pallas as pl
from jax.experimental.pallas import tpu as pltpu
from jax.experimental.pallas import tpu_sc as plsc

N = 10000
E = 320000
IN_CH = 128
D = 64
K = 3
ALPHA = 0.1

NC = 2
NS = 16
NW = NC * NS
L = 16

N_PAD = 10240
RPT = N_PAD // NS
B = 128
CH = 80
CHX = CH + 2
E_TILE = B * CH
E_PAD = E_TILE * NW

ROWBLK = 1024


def _mesh():
    return plsc.VectorSubcoreMesh(
        core_axis_name="c", subcore_axis_name="s",
        num_cores=NC, num_subcores=NS)


_SC_PARAMS = pltpu.CompilerParams(use_tc_tiling_on_sc=False)


@functools.partial(
    pl.kernel,
    out_type=jax.ShapeDtypeStruct((NC, 2, N_PAD), jnp.float32),
    mesh=_mesh(),
    compiler_params=_SC_PARAMS,
    scratch_types=[
        pltpu.VMEM((CHX, B), jnp.int32),
        pltpu.VMEM((CHX, B), jnp.int32),
        pltpu.VMEM((B,), jnp.float32),
        pltpu.VMEM((RPT,), jnp.float32),
        pltpu.VMEM_SHARED((N_PAD,), jnp.float32),
        pltpu.VMEM_SHARED((N_PAD,), jnp.float32),
    ],
)
def _deg_kernel(src_h, dst_h, out_h, src_v, dst_v, ones_v, zero_v,
                dego_sh, degi_sh):
    cid = lax.axis_index("c")
    sid = lax.axis_index("s")
    wid = cid * NS + sid

    def zi(i, c):
        zero_v[pl.ds(i * L, L)] = jnp.zeros((L,), jnp.float32)
        return c
    lax.fori_loop(0, RPT // L, zi, 0)

    def oi(i, c):
        ones_v[pl.ds(i * L, L)] = jnp.ones((L,), jnp.float32)
        return c
    lax.fori_loop(0, B // L, oi, 0)

    pltpu.sync_copy(zero_v, dego_sh.at[pl.ds(sid * RPT, RPT)])
    pltpu.sync_copy(zero_v, degi_sh.at[pl.ds(sid * RPT, RPT)])
    pltpu.sync_copy(src_h.at[wid], src_v)
    pltpu.sync_copy(dst_h.at[wid], dst_v)
    plsc.subcore_barrier()

    def body(j, c):
        pltpu.sync_copy(ones_v, dego_sh.at[src_v.at[j]], add=True)
        pltpu.sync_copy(ones_v, degi_sh.at[dst_v.at[j]], add=True)
        return c
    lax.fori_loop(0, CH, body, 0)

    plsc.subcore_barrier()
    sl = pl.ds(sid * RPT, RPT)
    pltpu.sync_copy(dego_sh.at[sl], out_h.at[cid, 0, sl])
    pltpu.sync_copy(degi_sh.at[sl], out_h.at[cid, 1, sl])


@functools.partial(
    pl.kernel,
    out_type=jax.ShapeDtypeStruct((NW, CH, B), jnp.float32),
    mesh=_mesh(),
    compiler_params=_SC_PARAMS,
    scratch_types=[
        pltpu.VMEM((CHX, B), jnp.int32),
        pltpu.VMEM((CHX, B), jnp.int32),
        pltpu.VMEM((CH, B), jnp.float32),
        pltpu.VMEM((CH, B), jnp.float32),
        pltpu.VMEM((B,), jnp.float32),
        pltpu.VMEM((B,), jnp.float32),
        pltpu.VMEM((B,), jnp.float32),
        pltpu.VMEM((B,), jnp.float32),
        pltpu.SemaphoreType.DMA,
        pltpu.SemaphoreType.DMA,
    ],
)
def _coef_kernel(src_h, dst_h, ew_h, no_h, ni_h, out_h,
                 src_v, dst_v, ew_v, coef_v,
                 nog0, nig0, nog1, nig1, sem0, sem1):
    cid = lax.axis_index("c")
    sid = lax.axis_index("s")
    wid = cid * NS + sid
    pltpu.sync_copy(src_h.at[wid], src_v)
    pltpu.sync_copy(dst_h.at[wid], dst_v)
    pltpu.sync_copy(ew_h.at[wid], ew_v)

    pltpu.async_copy(no_h.at[src_v.at[0]], nog0, sem0)
    pltpu.async_copy(ni_h.at[dst_v.at[0]], nig0, sem0)
    pltpu.async_copy(no_h.at[src_v.at[1]], nog1, sem1)
    pltpu.async_copy(ni_h.at[dst_v.at[1]], nig1, sem1)

    def half(t, nog, nig, sem):
        pltpu.make_async_copy(no_h.at[src_v.at[t]], nog, sem).wait()
        pltpu.make_async_copy(ni_h.at[dst_v.at[t]], nig, sem).wait()
        for g in range(B // L):
            sl = pl.ds(g * L, L)
            coef_v[t, sl] = nog[sl] * ew_v[t, sl] * nig[sl]
        pltpu.async_copy(no_h.at[src_v.at[t + 2]], nog, sem)
        pltpu.async_copy(ni_h.at[dst_v.at[t + 2]], nig, sem)

    def body(i, c):
        t = 2 * i
        half(t, nog0, nig0, sem0)
        half(t + 1, nog1, nig1, sem1)
        return c
    lax.fori_loop(0, CH // 2, body, 0)

    pltpu.make_async_copy(no_h.at[src_v.at[0]], nog0, sem0).wait()
    pltpu.make_async_copy(ni_h.at[dst_v.at[0]], nig0, sem0).wait()
    pltpu.make_async_copy(no_h.at[src_v.at[1]], nog1, sem1).wait()
    pltpu.make_async_copy(ni_h.at[dst_v.at[1]], nig1, sem1).wait()
    pltpu.sync_copy(coef_v, out_h.at[wid])


CB = 8


@functools.partial(
    pl.kernel,
    out_type=jax.ShapeDtypeStruct((NC, N_PAD, D), jnp.float32),
    mesh=_mesh(),
    compiler_params=_SC_PARAMS,
    scratch_types=[
        pltpu.VMEM((CHX, B), jnp.int32),
        pltpu.VMEM((CHX, B), jnp.int32),
        pltpu.VMEM((CB, B, L), jnp.float32),
        pltpu.VMEM((B, D), jnp.float32),
        pltpu.VMEM((B, D), jnp.float32),
        pltpu.VMEM_SHARED((N_PAD, D), jnp.float32),
        pltpu.SemaphoreType.DMA,
    ],
)
def _prop_kernel(h_h, src_h, dst_h, coefx_h, out_h,
                 src_v, dst_v, coefx_v, rows_v, zero_v, agg_sh, sem):
    cid = lax.axis_index("c")
    sid = lax.axis_index("s")
    wid = cid * NS + sid

    def zi(i, c):
        zero_v[i // (D // L), pl.ds((i % (D // L)) * L, L)] = (
            jnp.zeros((L,), jnp.float32))
        return c
    lax.fori_loop(0, B * D // L, zi, 0)

    def zc(i, c):
        pltpu.sync_copy(zero_v, agg_sh.at[pl.ds(sid * RPT + i * B, B)])
        return c
    lax.fori_loop(0, RPT // B, zc, 0)

    pltpu.sync_copy(src_h.at[wid], src_v)
    pltpu.sync_copy(dst_h.at[wid], dst_v)
    plsc.subcore_barrier()

    def blk(bI, c):
        pltpu.sync_copy(coefx_h.at[wid].at[pl.ds(bI * CB, CB)], coefx_v)

        def body(bi, cc):
            t = bI * CB + bi
            pltpu.async_copy(h_h.at[src_v.at[t]], rows_v, sem).wait()

            def scale(g, ccc):
                for m in range(L):
                    e = g * L + m
                    cvx = coefx_v[bi, e]
                    for k in range(D // L):
                        rows_v[e, pl.ds(k * L, L)] = (
                            rows_v[e, pl.ds(k * L, L)] * cvx)
                return ccc
            lax.fori_loop(0, B // L, scale, 0)
            pltpu.sync_copy(rows_v, agg_sh.at[dst_v.at[t]], add=True)
            return cc
        lax.fori_loop(0, CB, body, 0)
        return c
    lax.fori_loop(0, CH // CB, blk, 0)

    plsc.subcore_barrier()
    sl = pl.ds(sid * RPT, RPT)
    pltpu.sync_copy(agg_sh.at[sl], out_h.at[cid, sl])


def _mlp_in_call(x_p, w, b2):
    def body(x_r, w_r, b_r, o_r):
        o_r[...] = jnp.maximum(x_r[...] @ w_r[...] + b_r[...], 0.0)
    return pl.pallas_call(
        body,
        grid=(N_PAD // ROWBLK,),
        in_specs=[
            pl.BlockSpec((ROWBLK, IN_CH), lambda i: (i, 0)),
            pl.BlockSpec((IN_CH, D), lambda i: (0, 0)),
            pl.BlockSpec((1, D), lambda i: (0, 0)),
        ],
        out_specs=pl.BlockSpec((ROWBLK, D), lambda i: (i, 0)),
        out_shape=jax.ShapeDtypeStruct((N_PAD, D), jnp.float32),
    )(x_p, w, b2)


def _expand_call(coef2d, expmat):
    def body(c_r, m_r, o_r):
        o_r[...] = c_r[...] @ m_r[...]
    R = NW * CH
    BLK = 512
    return pl.pallas_call(
        body,
        grid=(R // BLK,),
        in_specs=[
            pl.BlockSpec((BLK, B), lambda i: (i, 0)),
            pl.BlockSpec((B, B * L), lambda i: (0, 0)),
        ],
        out_specs=pl.BlockSpec((BLK, B * L), lambda i: (i, 0)),
        out_shape=jax.ShapeDtypeStruct((R, B * L), jnp.float32),
    )(coef2d, expmat)


_DEG_ROWS = 2 * 2 * N_PAD // 128
_NR = N_PAD // 128


def _norm_call(deg_flat):
    def body(d_r, o_r):
        d = d_r[...]
        tot_o = d[0:_NR] + d[2 * _NR:3 * _NR]
        tot_i = d[_NR:2 * _NR] + d[3 * _NR:4 * _NR]
        no = jnp.where(tot_o > 0, lax.rsqrt(tot_o), 0.0)
        ni = jnp.where(tot_i > 0, lax.rsqrt(tot_i), 0.0)
        o_r[...] = jnp.concatenate([no, ni], axis=0)
    return pl.pallas_call(
        body,
        out_shape=jax.ShapeDtypeStruct((2 * _NR, 128), jnp.float32),
    )(deg_flat)


def _combine_call(part, feat0):
    def body(p_r, f_r, o_r):
        p = p_r[...]
        o_r[...] = (1.0 - ALPHA) * (p[0] + p[1]) + ALPHA * f_r[...]
    return pl.pallas_call(
        body,
        grid=(N_PAD // ROWBLK,),
        in_specs=[
            pl.BlockSpec((NC, ROWBLK, D), lambda i: (0, i, 0)),
            pl.BlockSpec((ROWBLK, D), lambda i: (i, 0)),
        ],
        out_specs=pl.BlockSpec((ROWBLK, D), lambda i: (i, 0)),
        out_shape=jax.ShapeDtypeStruct((N_PAD, D), jnp.float32),
    )(part, feat0)


def _final_call(part, feat0, w, b2):
    def body(p_r, f_r, w_r, b_r, o_r):
        p = p_r[...]
        h = (1.0 - ALPHA) * (p[0] + p[1]) + ALPHA * f_r[...]
        o_r[...] = h @ w_r[...] + b_r[...]
    return pl.pallas_call(
        body,
        grid=(N_PAD // ROWBLK,),
        in_specs=[
            pl.BlockSpec((NC, ROWBLK, D), lambda i: (0, i, 0)),
            pl.BlockSpec((ROWBLK, D), lambda i: (i, 0)),
            pl.BlockSpec((D, D), lambda i: (0, 0)),
            pl.BlockSpec((1, D), lambda i: (0, 0)),
        ],
        out_specs=pl.BlockSpec((ROWBLK, D), lambda i: (i, 0)),
        out_shape=jax.ShapeDtypeStruct((N_PAD, D), jnp.float32),
    )(part, feat0, w, b2)


def _extend(a, fill):
    a_p = jnp.concatenate(
        [a, jnp.full((E_PAD - E,), fill, a.dtype)]).reshape(NW, CH, B)
    return jnp.concatenate(
        [a_p, jnp.full((NW, 2, B), fill, a.dtype)], axis=1)


def kernel(x, edge_index, edge_weight, W_in, b_in, W_out, b_out):
    src_x = _extend(edge_index[0], N)
    dst_x = _extend(edge_index[1], N)
    ew_p = jnp.concatenate(
        [edge_weight, jnp.zeros((E_PAD - E,), jnp.float32)]).reshape(NW, CH, B)
    x_p = jnp.pad(x, ((0, N_PAD - N), (0, 0)))

    deg = _deg_kernel(src_x, dst_x)
    norms = _norm_call(deg.reshape(_DEG_ROWS, 128))
    no = norms[:_NR].reshape(N_PAD)
    ni = norms[_NR:].reshape(N_PAD)

    h0 = _mlp_in_call(x_p, W_in, b_in.reshape(1, D))
    coef = _coef_kernel(src_x, dst_x, ew_p, no, ni)

    expmat = jnp.zeros((B, B * L), jnp.float32).at[
        jnp.arange(B).repeat(L), jnp.arange(B * L)].set(1.0)
    coefx = _expand_call(coef.reshape(NW * CH, B), expmat).reshape(
        NW, CH, B, L)

    h = h0
    out = None
    for t in range(K):
        part = _prop_kernel(h, src_x, dst_x, coefx)
        if t < K - 1:
            h = _combine_call(part, h0)
        else:
            out = _final_call(part, h0, W_out, b_out.reshape(1, D))
    return out[:N]


# speedup vs baseline: 1.3574x; 1.1876x over previous
import functools

import jax
import jax.numpy as jnp
from jax import lax
from jax.experimental import pallas as pl
from jax.experimental.pallas import tpu as pltpu
from jax.experimental.pallas import tpu_sc as plsc

N = 10000
E = 320000
IN_CH = 128
D = 64
K = 3
ALPHA = 0.1

NC = 2
NS = 16
NW = NC * NS
L = 16

N_PAD = 10240
RPT = N_PAD // NS
B = 128
CH = 80
TB = NW * CH
TBX = TB + 64
E_PAD = TB * B

FB = 112
SB = 2 * CH - FB
KB = 4

ROWBLK = 1024


def _mesh():
    return plsc.VectorSubcoreMesh(
        core_axis_name="c", subcore_axis_name="s",
        num_cores=NC, num_subcores=NS)


_SC_PARAMS = pltpu.CompilerParams(use_tc_tiling_on_sc=False)


@functools.partial(
    pl.kernel,
    out_type=jax.ShapeDtypeStruct((NC, 2, N_PAD), jnp.float32),
    mesh=_mesh(),
    compiler_params=_SC_PARAMS,
    scratch_types=[
        pltpu.VMEM((CH, B), jnp.int32),
        pltpu.VMEM((CH, B), jnp.int32),
        pltpu.VMEM((B,), jnp.float32),
        pltpu.VMEM((RPT,), jnp.float32),
        pltpu.VMEM_SHARED((N_PAD,), jnp.float32),
        pltpu.VMEM_SHARED((N_PAD,), jnp.float32),
    ],
)
def _deg_kernel(src_h, dst_h, out_h, src_v, dst_v, ones_v, zero_v,
                dego_sh, degi_sh):
    cid = lax.axis_index("c")
    sid = lax.axis_index("s")
    wid = cid * NS + sid

    def zi(i, c):
        zero_v[pl.ds(i * L, L)] = jnp.zeros((L,), jnp.float32)
        return c
    lax.fori_loop(0, RPT // L, zi, 0)

    def oi(i, c):
        ones_v[pl.ds(i * L, L)] = jnp.ones((L,), jnp.float32)
        return c
    lax.fori_loop(0, B // L, oi, 0)

    pltpu.sync_copy(zero_v, dego_sh.at[pl.ds(sid * RPT, RPT)])
    pltpu.sync_copy(zero_v, degi_sh.at[pl.ds(sid * RPT, RPT)])
    pltpu.sync_copy(src_h.at[pl.ds(wid * CH, CH)], src_v)
    pltpu.sync_copy(dst_h.at[pl.ds(wid * CH, CH)], dst_v)
    plsc.subcore_barrier()

    def body(j, c):
        pltpu.sync_copy(ones_v, dego_sh.at[src_v.at[j]], add=True)
        pltpu.sync_copy(ones_v, degi_sh.at[dst_v.at[j]], add=True)
        return c
    lax.fori_loop(0, CH, body, 0)

    plsc.subcore_barrier()
    sl = pl.ds(sid * RPT, RPT)
    pltpu.sync_copy(dego_sh.at[sl], out_h.at[cid, 0, sl])
    pltpu.sync_copy(degi_sh.at[sl], out_h.at[cid, 1, sl])


@functools.partial(
    pl.kernel,
    out_type=jax.ShapeDtypeStruct((TB, B), jnp.float32),
    mesh=_mesh(),
    compiler_params=_SC_PARAMS,
    scratch_types=[
        pltpu.VMEM((CH + 2, B), jnp.int32),
        pltpu.VMEM((CH + 2, B), jnp.int32),
        pltpu.VMEM((CH, B), jnp.float32),
        pltpu.VMEM((CH, B), jnp.float32),
        pltpu.VMEM((B,), jnp.float32),
        pltpu.VMEM((B,), jnp.float32),
        pltpu.VMEM((B,), jnp.float32),
        pltpu.VMEM((B,), jnp.float32),
        pltpu.SemaphoreType.DMA,
        pltpu.SemaphoreType.DMA,
    ],
)
def _coef_kernel(src_h, dst_h, ew_h, no_h, ni_h, out_h,
                 src_v, dst_v, ew_v, coef_v,
                 nog0, nig0, nog1, nig1, sem0, sem1):
    cid = lax.axis_index("c")
    sid = lax.axis_index("s")
    wid = cid * NS + sid
    pltpu.sync_copy(src_h.at[pl.ds(wid * CH, CH + 2)], src_v)
    pltpu.sync_copy(dst_h.at[pl.ds(wid * CH, CH + 2)], dst_v)
    pltpu.sync_copy(ew_h.at[pl.ds(wid * CH, CH)], ew_v)

    pltpu.async_copy(no_h.at[src_v.at[0]], nog0, sem0)
    pltpu.async_copy(ni_h.at[dst_v.at[0]], nig0, sem0)
    pltpu.async_copy(no_h.at[src_v.at[1]], nog1, sem1)
    pltpu.async_copy(ni_h.at[dst_v.at[1]], nig1, sem1)

    def half(t, nog, nig, sem):
        pltpu.make_async_copy(no_h.at[src_v.at[t]], nog, sem).wait()
        pltpu.make_async_copy(ni_h.at[dst_v.at[t]], nig, sem).wait()
        for g in range(B // L):
            sl = pl.ds(g * L, L)
            coef_v[t, sl] = nog[sl] * ew_v[t, sl] * nig[sl]
        pltpu.async_copy(no_h.at[src_v.at[t + 2]], nog, sem)
        pltpu.async_copy(ni_h.at[dst_v.at[t + 2]], nig, sem)

    def body(i, c):
        t = 2 * i
        half(t, nog0, nig0, sem0)
        half(t + 1, nog1, nig1, sem1)
        return c
    lax.fori_loop(0, CH // 2, body, 0)

    pltpu.make_async_copy(no_h.at[src_v.at[0]], nog0, sem0).wait()
    pltpu.make_async_copy(ni_h.at[dst_v.at[0]], nig0, sem0).wait()
    pltpu.make_async_copy(no_h.at[src_v.at[1]], nog1, sem1).wait()
    pltpu.make_async_copy(ni_h.at[dst_v.at[1]], nig1, sem1).wait()
    pltpu.sync_copy(coef_v, out_h.at[pl.ds(wid * CH, CH)])


@functools.partial(
    pl.kernel,
    out_type=jax.ShapeDtypeStruct((NC, N_PAD, D), jnp.float32),
    mesh=_mesh(),
    compiler_params=_SC_PARAMS,
    scratch_types=[
        pltpu.VMEM((FB, B), jnp.int32),
        pltpu.VMEM((FB, B), jnp.int32),
        pltpu.VMEM((KB, B, L), jnp.float32),
        [pltpu.VMEM((B, D), jnp.float32)] * KB,
        pltpu.VMEM_SHARED((N_PAD, D), jnp.float32),
        pltpu.SemaphoreType.DMA,
    ],
)
def _prop_kernel(h_h, src_h, dst_h, coefx_h, out_h,
                 src_v, dst_v, coefx_v, bufs, agg_sh, gsem):
    cid = lax.axis_index("c")
    sid = lax.axis_index("s")

    base_b = jnp.where(cid == 0, sid * FB, NS * FB + sid * SB)
    nblk = jnp.where(cid == 0, FB // KB, SB // KB)

    zero_v = bufs[0]

    def zi(i, c):
        zero_v[i // (D // L), pl.ds((i % (D // L)) * L, L)] = (
            jnp.zeros((L,), jnp.float32))
        return c
    lax.fori_loop(0, B * D // L, zi, 0)

    def zc(i, c):
        pltpu.sync_copy(zero_v, agg_sh.at[pl.ds(sid * RPT + i * B, B)])
        return c
    lax.fori_loop(0, RPT // B, zc, 0)

    pltpu.sync_copy(src_h.at[pl.ds(base_b, FB)], src_v)
    pltpu.sync_copy(dst_h.at[pl.ds(base_b, FB)], dst_v)
    plsc.subcore_barrier()

    def blk(bI, c):
        t0 = bI * KB
        for b in range(KB):
            pltpu.async_copy(h_h.at[src_v.at[t0 + b]], bufs[b], gsem)
        pltpu.sync_copy(coefx_h.at[pl.ds(base_b + t0, KB)], coefx_v)
        for b in range(KB):
            pltpu.make_async_copy(
                h_h.at[src_v.at[t0 + b]], bufs[b], gsem).wait()

        for b in range(KB):
            buf = bufs[b]

            def scale(g, ccc):
                for m in range(L):
                    e = g * L + m
                    cvx = coefx_v[b, e]
                    for k in range(D // L):
                        buf[e, pl.ds(k * L, L)] = (
                            buf[e, pl.ds(k * L, L)] * cvx)
                return ccc
            lax.fori_loop(0, B // L, scale, 0)
            pltpu.sync_copy(buf, agg_sh.at[dst_v.at[t0 + b]], add=True)
        return c
    lax.fori_loop(0, nblk, blk, 0)

    plsc.subcore_barrier()
    sl = pl.ds(sid * RPT, RPT)
    pltpu.sync_copy(agg_sh.at[sl], out_h.at[cid, sl])


def _mlp_in_call(x_p, w, b2):
    def body(x_r, w_r, b_r, o_r):
        o_r[...] = jnp.maximum(x_r[...] @ w_r[...] + b_r[...], 0.0)
    return pl.pallas_call(
        body,
        grid=(N_PAD // ROWBLK,),
        in_specs=[
            pl.BlockSpec((ROWBLK, IN_CH), lambda i: (i, 0)),
            pl.BlockSpec((IN_CH, D), lambda i: (0, 0)),
            pl.BlockSpec((1, D), lambda i: (0, 0)),
        ],
        out_specs=pl.BlockSpec((ROWBLK, D), lambda i: (i, 0)),
        out_shape=jax.ShapeDtypeStruct((N_PAD, D), jnp.float32),
    )(x_p, w, b2)


def _expand_call(coef2d, expmat):
    def body(c_r, m_r, o_r):
        o_r[...] = jax.lax.dot(c_r[...], m_r[...],
                               precision=lax.Precision.HIGHEST)
    BLK = 512
    return pl.pallas_call(
        body,
        grid=(TB // BLK,),
        in_specs=[
            pl.BlockSpec((BLK, B), lambda i: (i, 0)),
            pl.BlockSpec((B, B * L), lambda i: (0, 0)),
        ],
        out_specs=pl.BlockSpec((BLK, B * L), lambda i: (i, 0)),
        out_shape=jax.ShapeDtypeStruct((TB, B * L), jnp.float32),
    )(coef2d, expmat)


_DEG_ROWS = 2 * 2 * N_PAD // 128
_NR = N_PAD // 128


def _norm_call(deg_flat):
    def body(d_r, o_r):
        d = d_r[...]
        tot_o = d[0:_NR] + d[2 * _NR:3 * _NR]
        tot_i = d[_NR:2 * _NR] + d[3 * _NR:4 * _NR]
        no = jnp.where(tot_o > 0, lax.rsqrt(tot_o), 0.0)
        ni = jnp.where(tot_i > 0, lax.rsqrt(tot_i), 0.0)
        o_r[...] = jnp.concatenate([no, ni], axis=0)
    return pl.pallas_call(
        body,
        out_shape=jax.ShapeDtypeStruct((2 * _NR, 128), jnp.float32),
    )(deg_flat)


def _combine_call(part, feat0):
    def body(p_r, f_r, o_r):
        p = p_r[...]
        o_r[...] = (1.0 - ALPHA) * (p[0] + p[1]) + ALPHA * f_r[...]
    return pl.pallas_call(
        body,
        grid=(N_PAD // ROWBLK,),
        in_specs=[
            pl.BlockSpec((NC, ROWBLK, D), lambda i: (0, i, 0)),
            pl.BlockSpec((ROWBLK, D), lambda i: (i, 0)),
        ],
        out_specs=pl.BlockSpec((ROWBLK, D), lambda i: (i, 0)),
        out_shape=jax.ShapeDtypeStruct((N_PAD, D), jnp.float32),
    )(part, feat0)


def _final_call(part, feat0, w, b2):
    def body(p_r, f_r, w_r, b_r, o_r):
        p = p_r[...]
        h = (1.0 - ALPHA) * (p[0] + p[1]) + ALPHA * f_r[...]
        o_r[...] = h @ w_r[...] + b_r[...]
    return pl.pallas_call(
        body,
        grid=(N_PAD // ROWBLK,),
        in_specs=[
            pl.BlockSpec((NC, ROWBLK, D), lambda i: (0, i, 0)),
            pl.BlockSpec((ROWBLK, D), lambda i: (i, 0)),
            pl.BlockSpec((D, D), lambda i: (0, 0)),
            pl.BlockSpec((1, D), lambda i: (0, 0)),
        ],
        out_specs=pl.BlockSpec((ROWBLK, D), lambda i: (i, 0)),
        out_shape=jax.ShapeDtypeStruct((N_PAD, D), jnp.float32),
    )(part, feat0, w, b2)


def _extend(a, fill):
    return jnp.concatenate(
        [a, jnp.full((TBX * B - E,), fill, a.dtype)]).reshape(TBX, B)


def kernel(x, edge_index, edge_weight, W_in, b_in, W_out, b_out):
    src_x = _extend(edge_index[0], N)
    dst_x = _extend(edge_index[1], N)
    ew_p = jnp.concatenate(
        [edge_weight, jnp.zeros((E_PAD - E,), jnp.float32)]).reshape(TB, B)
    x_p = jnp.pad(x, ((0, N_PAD - N), (0, 0)))

    deg = _deg_kernel(src_x, dst_x)
    norms = _norm_call(deg.reshape(_DEG_ROWS, 128))
    no = norms[:_NR].reshape(N_PAD)
    ni = norms[_NR:].reshape(N_PAD)

    h0 = _mlp_in_call(x_p, W_in, b_in.reshape(1, D))
    coef = _coef_kernel(src_x, dst_x, ew_p, no, ni)

    expmat = jnp.zeros((B, B * L), jnp.float32).at[
        jnp.arange(B).repeat(L), jnp.arange(B * L)].set(1.0)
    coefx = _expand_call(coef, expmat).reshape(TB, B, L)

    h = h0
    out = None
    for t in range(K):
        part = _prop_kernel(h, src_x, dst_x, coefx)
        if t < K - 1:
            h = _combine_call(part, h0)
        else:
            out = _final_call(part, h0, W_out, b_out.reshape(1, D))
    return out[:N]


# speedup vs baseline: 1.4295x; 1.0532x over previous
import functools

import jax
import jax.numpy as jnp
from jax import lax
from jax.experimental import pallas as pl
from jax.experimental.pallas import tpu as pltpu
from jax.experimental.pallas import tpu_sc as plsc

N = 10000
E = 320000
IN_CH = 128
D = 64
K = 3
ALPHA = 0.1

NC = 2
NS = 16
NW = NC * NS
L = 16

N_PAD = 10240
RPT = N_PAD // NS
B = 128
CH = 80
TB = NW * CH
TBX = TB + 64
E_PAD = TB * B

FB = 120
SB = 2 * CH - FB
KB = 4

ROWBLK = 1024


def _mesh():
    return plsc.VectorSubcoreMesh(
        core_axis_name="c", subcore_axis_name="s",
        num_cores=NC, num_subcores=NS)


_SC_PARAMS = pltpu.CompilerParams(use_tc_tiling_on_sc=False)


@functools.partial(
    pl.kernel,
    out_type=jax.ShapeDtypeStruct((NC, 2, N_PAD), jnp.float32),
    mesh=_mesh(),
    compiler_params=_SC_PARAMS,
    scratch_types=[
        pltpu.VMEM((CH, B), jnp.int32),
        pltpu.VMEM((CH, B), jnp.int32),
        pltpu.VMEM((B,), jnp.float32),
        pltpu.VMEM((RPT,), jnp.float32),
        pltpu.VMEM_SHARED((N_PAD,), jnp.float32),
        pltpu.VMEM_SHARED((N_PAD,), jnp.float32),
    ],
)
def _deg_kernel(src_h, dst_h, out_h, src_v, dst_v, ones_v, zero_v,
                dego_sh, degi_sh):
    cid = lax.axis_index("c")
    sid = lax.axis_index("s")
    wid = cid * NS + sid

    def zi(i, c):
        zero_v[pl.ds(i * L, L)] = jnp.zeros((L,), jnp.float32)
        return c
    lax.fori_loop(0, RPT // L, zi, 0)

    def oi(i, c):
        ones_v[pl.ds(i * L, L)] = jnp.ones((L,), jnp.float32)
        return c
    lax.fori_loop(0, B // L, oi, 0)

    pltpu.sync_copy(zero_v, dego_sh.at[pl.ds(sid * RPT, RPT)])
    pltpu.sync_copy(zero_v, degi_sh.at[pl.ds(sid * RPT, RPT)])
    pltpu.sync_copy(src_h.at[pl.ds(wid * CH, CH)], src_v)
    pltpu.sync_copy(dst_h.at[pl.ds(wid * CH, CH)], dst_v)
    plsc.subcore_barrier()

    def body(j, c):
        pltpu.sync_copy(ones_v, dego_sh.at[src_v.at[j]], add=True)
        pltpu.sync_copy(ones_v, degi_sh.at[dst_v.at[j]], add=True)
        return c
    lax.fori_loop(0, CH, body, 0)

    plsc.subcore_barrier()
    sl = pl.ds(sid * RPT, RPT)
    pltpu.sync_copy(dego_sh.at[sl], out_h.at[cid, 0, sl])
    pltpu.sync_copy(degi_sh.at[sl], out_h.at[cid, 1, sl])


@functools.partial(
    pl.kernel,
    out_type=jax.ShapeDtypeStruct((TB, B), jnp.float32),
    mesh=_mesh(),
    compiler_params=_SC_PARAMS,
    scratch_types=[
        pltpu.VMEM((CH + 2, B), jnp.int32),
        pltpu.VMEM((CH + 2, B), jnp.int32),
        pltpu.VMEM((CH, B), jnp.float32),
        pltpu.VMEM((CH, B), jnp.float32),
        pltpu.VMEM((B,), jnp.float32),
        pltpu.VMEM((B,), jnp.float32),
        pltpu.VMEM((B,), jnp.float32),
        pltpu.VMEM((B,), jnp.float32),
        pltpu.SemaphoreType.DMA,
        pltpu.SemaphoreType.DMA,
    ],
)
def _coef_kernel(src_h, dst_h, ew_h, no_h, ni_h, out_h,
                 src_v, dst_v, ew_v, coef_v,
                 nog0, nig0, nog1, nig1, sem0, sem1):
    cid = lax.axis_index("c")
    sid = lax.axis_index("s")
    wid = cid * NS + sid
    pltpu.sync_copy(src_h.at[pl.ds(wid * CH, CH + 2)], src_v)
    pltpu.sync_copy(dst_h.at[pl.ds(wid * CH, CH + 2)], dst_v)
    pltpu.sync_copy(ew_h.at[pl.ds(wid * CH, CH)], ew_v)

    pltpu.async_copy(no_h.at[src_v.at[0]], nog0, sem0)
    pltpu.async_copy(ni_h.at[dst_v.at[0]], nig0, sem0)
    pltpu.async_copy(no_h.at[src_v.at[1]], nog1, sem1)
    pltpu.async_copy(ni_h.at[dst_v.at[1]], nig1, sem1)

    def half(t, nog, nig, sem):
        pltpu.make_async_copy(no_h.at[src_v.at[t]], nog, sem).wait()
        pltpu.make_async_copy(ni_h.at[dst_v.at[t]], nig, sem).wait()
        for g in range(B // L):
            sl = pl.ds(g * L, L)
            coef_v[t, sl] = nog[sl] * ew_v[t, sl] * nig[sl]
        pltpu.async_copy(no_h.at[src_v.at[t + 2]], nog, sem)
        pltpu.async_copy(ni_h.at[dst_v.at[t + 2]], nig, sem)

    def body(i, c):
        t = 2 * i
        half(t, nog0, nig0, sem0)
        half(t + 1, nog1, nig1, sem1)
        return c
    lax.fori_loop(0, CH // 2, body, 0)

    pltpu.make_async_copy(no_h.at[src_v.at[0]], nog0, sem0).wait()
    pltpu.make_async_copy(ni_h.at[dst_v.at[0]], nig0, sem0).wait()
    pltpu.make_async_copy(no_h.at[src_v.at[1]], nog1, sem1).wait()
    pltpu.make_async_copy(ni_h.at[dst_v.at[1]], nig1, sem1).wait()
    pltpu.sync_copy(coef_v, out_h.at[pl.ds(wid * CH, CH)])


@functools.partial(
    pl.kernel,
    out_type=jax.ShapeDtypeStruct((NC, N_PAD, D), jnp.float32),
    mesh=_mesh(),
    compiler_params=_SC_PARAMS,
    scratch_types=[
        pltpu.VMEM((FB, B), jnp.int32),
        pltpu.VMEM((FB, B), jnp.int32),
        pltpu.VMEM((KB, B, L), jnp.float32),
        [pltpu.VMEM((B, D), jnp.float32)] * KB,
        pltpu.VMEM_SHARED((N_PAD, D), jnp.float32),
        pltpu.SemaphoreType.DMA,
    ],
)
def _prop_kernel(h_h, src_h, dst_h, coefx_h, out_h,
                 src_v, dst_v, coefx_v, bufs, agg_sh, gsem):
    cid = lax.axis_index("c")
    sid = lax.axis_index("s")

    base_b = jnp.where(cid == 0, sid * FB, NS * FB + sid * SB)
    nblk = jnp.where(cid == 0, FB // KB, SB // KB)

    zero_v = bufs[0]

    def zi(i, c):
        zero_v[i // (D // L), pl.ds((i % (D // L)) * L, L)] = (
            jnp.zeros((L,), jnp.float32))
        return c
    lax.fori_loop(0, B * D // L, zi, 0)

    def zc(i, c):
        pltpu.sync_copy(zero_v, agg_sh.at[pl.ds(sid * RPT + i * B, B)])
        return c
    lax.fori_loop(0, RPT // B, zc, 0)

    pltpu.sync_copy(src_h.at[pl.ds(base_b, FB)], src_v)
    pltpu.sync_copy(dst_h.at[pl.ds(base_b, FB)], dst_v)
    plsc.subcore_barrier()

    def blk(bI, c):
        t0 = bI * KB
        for b in range(KB):
            pltpu.async_copy(h_h.at[src_v.at[t0 + b]], bufs[b], gsem)
        pltpu.sync_copy(coefx_h.at[pl.ds(base_b + t0, KB)], coefx_v)
        for b in range(KB):
            pltpu.make_async_copy(
                h_h.at[src_v.at[t0 + b]], bufs[b], gsem).wait()

        for b in range(KB):
            buf = bufs[b]

            def scale(g, ccc):
                for m in range(L):
                    e = g * L + m
                    cvx = coefx_v[b, e]
                    for k in range(D // L):
                        buf[e, pl.ds(k * L, L)] = (
                            buf[e, pl.ds(k * L, L)] * cvx)
                return ccc
            lax.fori_loop(0, B // L, scale, 0)
            pltpu.sync_copy(buf, agg_sh.at[dst_v.at[t0 + b]], add=True)
        return c
    lax.fori_loop(0, nblk, blk, 0)

    plsc.subcore_barrier()
    sl = pl.ds(sid * RPT, RPT)
    pltpu.sync_copy(agg_sh.at[sl], out_h.at[cid, sl])


def _mlp_in_call(x_p, w, b2):
    def body(x_r, w_r, b_r, o_r):
        o_r[...] = jnp.maximum(x_r[...] @ w_r[...] + b_r[...], 0.0)
    return pl.pallas_call(
        body,
        grid=(N_PAD // ROWBLK,),
        in_specs=[
            pl.BlockSpec((ROWBLK, IN_CH), lambda i: (i, 0)),
            pl.BlockSpec((IN_CH, D), lambda i: (0, 0)),
            pl.BlockSpec((1, D), lambda i: (0, 0)),
        ],
        out_specs=pl.BlockSpec((ROWBLK, D), lambda i: (i, 0)),
        out_shape=jax.ShapeDtypeStruct((N_PAD, D), jnp.float32),
    )(x_p, w, b2)


def _expand_call(coef2d, expmat):
    def body(c_r, m_r, o_r):
        o_r[...] = jax.lax.dot(c_r[...], m_r[...],
                               precision=lax.Precision.HIGHEST)
    BLK = 512
    return pl.pallas_call(
        body,
        grid=(TB // BLK,),
        in_specs=[
            pl.BlockSpec((BLK, B), lambda i: (i, 0)),
            pl.BlockSpec((B, B * L), lambda i: (0, 0)),
        ],
        out_specs=pl.BlockSpec((BLK, B * L), lambda i: (i, 0)),
        out_shape=jax.ShapeDtypeStruct((TB, B * L), jnp.float32),
    )(coef2d, expmat)


_DEG_ROWS = 2 * 2 * N_PAD // 128
_NR = N_PAD // 128


def _norm_call(deg_flat):
    def body(d_r, o_r):
        d = d_r[...]
        tot_o = d[0:_NR] + d[2 * _NR:3 * _NR]
        tot_i = d[_NR:2 * _NR] + d[3 * _NR:4 * _NR]
        no = jnp.where(tot_o > 0, lax.rsqrt(tot_o), 0.0)
        ni = jnp.where(tot_i > 0, lax.rsqrt(tot_i), 0.0)
        o_r[...] = jnp.concatenate([no, ni], axis=0)
    return pl.pallas_call(
        body,
        out_shape=jax.ShapeDtypeStruct((2 * _NR, 128), jnp.float32),
    )(deg_flat)


def _combine_call(part, feat0):
    def body(p_r, f_r, o_r):
        p = p_r[...]
        o_r[...] = (1.0 - ALPHA) * (p[0] + p[1]) + ALPHA * f_r[...]
    return pl.pallas_call(
        body,
        grid=(N_PAD // ROWBLK,),
        in_specs=[
            pl.BlockSpec((NC, ROWBLK, D), lambda i: (0, i, 0)),
            pl.BlockSpec((ROWBLK, D), lambda i: (i, 0)),
        ],
        out_specs=pl.BlockSpec((ROWBLK, D), lambda i: (i, 0)),
        out_shape=jax.ShapeDtypeStruct((N_PAD, D), jnp.float32),
    )(part, feat0)


def _final_call(part, feat0, w, b2):
    def body(p_r, f_r, w_r, b_r, o_r):
        p = p_r[...]
        h = (1.0 - ALPHA) * (p[0] + p[1]) + ALPHA * f_r[...]
        o_r[...] = h @ w_r[...] + b_r[...]
    return pl.pallas_call(
        body,
        grid=(N_PAD // ROWBLK,),
        in_specs=[
            pl.BlockSpec((NC, ROWBLK, D), lambda i: (0, i, 0)),
            pl.BlockSpec((ROWBLK, D), lambda i: (i, 0)),
            pl.BlockSpec((D, D), lambda i: (0, 0)),
            pl.BlockSpec((1, D), lambda i: (0, 0)),
        ],
        out_specs=pl.BlockSpec((ROWBLK, D), lambda i: (i, 0)),
        out_shape=jax.ShapeDtypeStruct((N_PAD, D), jnp.float32),
    )(part, feat0, w, b2)


def _extend(a, fill):
    return jnp.concatenate(
        [a, jnp.full((TBX * B - E,), fill, a.dtype)]).reshape(TBX, B)


def kernel(x, edge_index, edge_weight, W_in, b_in, W_out, b_out):
    src_x = _extend(edge_index[0], N)
    dst_x = _extend(edge_index[1], N)
    ew_p = jnp.concatenate(
        [edge_weight, jnp.zeros((E_PAD - E,), jnp.float32)]).reshape(TB, B)
    x_p = jnp.pad(x, ((0, N_PAD - N), (0, 0)))

    deg = _deg_kernel(src_x, dst_x)
    norms = _norm_call(deg.reshape(_DEG_ROWS, 128))
    no = norms[:_NR].reshape(N_PAD)
    ni = norms[_NR:].reshape(N_PAD)

    h0 = _mlp_in_call(x_p, W_in, b_in.reshape(1, D))
    coef = _coef_kernel(src_x, dst_x, ew_p, no, ni)

    expmat = jnp.zeros((B, B * L), jnp.float32).at[
        jnp.arange(B).repeat(L), jnp.arange(B * L)].set(1.0)
    coefx = _expand_call(coef, expmat).reshape(TB, B, L)

    h = h0
    out = None
    for t in range(K):
        part = _prop_kernel(h, src_x, dst_x, coefx)
        if t < K - 1:
            h = _combine_call(part, h0)
        else:
            out = _final_call(part, h0, W_out, b_out.reshape(1, D))
    return out[:N]


# speedup vs baseline: 1.4442x; 1.0102x over previous
import functools

import jax
import jax.numpy as jnp
from jax import lax
from jax.experimental import pallas as pl
from jax.experimental.pallas import tpu as pltpu
from jax.experimental.pallas import tpu_sc as plsc

N = 10000
E = 320000
IN_CH = 128
D = 64
K = 3
ALPHA = 0.1

NC = 2
NS = 16
NW = NC * NS
L = 16

N_PAD = 10240
RPT = N_PAD // NS
B = 128
CH = 80
TB = NW * CH
TBX = TB + 64
E_PAD = TB * B

FB = 124
SB = 2 * CH - FB
KB = 4

ROWBLK = 1024


def _mesh():
    return plsc.VectorSubcoreMesh(
        core_axis_name="c", subcore_axis_name="s",
        num_cores=NC, num_subcores=NS)


_SC_PARAMS = pltpu.CompilerParams(use_tc_tiling_on_sc=False)


@functools.partial(
    pl.kernel,
    out_type=jax.ShapeDtypeStruct((NC, 2, N_PAD), jnp.float32),
    mesh=_mesh(),
    compiler_params=_SC_PARAMS,
    scratch_types=[
        pltpu.VMEM((CH, B), jnp.int32),
        pltpu.VMEM((CH, B), jnp.int32),
        pltpu.VMEM((B,), jnp.float32),
        pltpu.VMEM((RPT,), jnp.float32),
        pltpu.VMEM_SHARED((N_PAD,), jnp.float32),
        pltpu.VMEM_SHARED((N_PAD,), jnp.float32),
    ],
)
def _deg_kernel(src_h, dst_h, out_h, src_v, dst_v, ones_v, zero_v,
                dego_sh, degi_sh):
    cid = lax.axis_index("c")
    sid = lax.axis_index("s")
    wid = cid * NS + sid

    def zi(i, c):
        zero_v[pl.ds(i * L, L)] = jnp.zeros((L,), jnp.float32)
        return c
    lax.fori_loop(0, RPT // L, zi, 0)

    def oi(i, c):
        ones_v[pl.ds(i * L, L)] = jnp.ones((L,), jnp.float32)
        return c
    lax.fori_loop(0, B // L, oi, 0)

    pltpu.sync_copy(zero_v, dego_sh.at[pl.ds(sid * RPT, RPT)])
    pltpu.sync_copy(zero_v, degi_sh.at[pl.ds(sid * RPT, RPT)])
    pltpu.sync_copy(src_h.at[pl.ds(wid * CH, CH)], src_v)
    pltpu.sync_copy(dst_h.at[pl.ds(wid * CH, CH)], dst_v)
    plsc.subcore_barrier()

    def body(j, c):
        pltpu.sync_copy(ones_v, dego_sh.at[src_v.at[j]], add=True)
        pltpu.sync_copy(ones_v, degi_sh.at[dst_v.at[j]], add=True)
        return c
    lax.fori_loop(0, CH, body, 0)

    plsc.subcore_barrier()
    sl = pl.ds(sid * RPT, RPT)
    pltpu.sync_copy(dego_sh.at[sl], out_h.at[cid, 0, sl])
    pltpu.sync_copy(degi_sh.at[sl], out_h.at[cid, 1, sl])


@functools.partial(
    pl.kernel,
    out_type=jax.ShapeDtypeStruct((TB, B), jnp.float32),
    mesh=_mesh(),
    compiler_params=_SC_PARAMS,
    scratch_types=[
        pltpu.VMEM((CH + 2, B), jnp.int32),
        pltpu.VMEM((CH + 2, B), jnp.int32),
        pltpu.VMEM((CH, B), jnp.float32),
        pltpu.VMEM((CH, B), jnp.float32),
        pltpu.VMEM((B,), jnp.float32),
        pltpu.VMEM((B,), jnp.float32),
        pltpu.VMEM((B,), jnp.float32),
        pltpu.VMEM((B,), jnp.float32),
        pltpu.SemaphoreType.DMA,
        pltpu.SemaphoreType.DMA,
    ],
)
def _coef_kernel(src_h, dst_h, ew_h, no_h, ni_h, out_h,
                 src_v, dst_v, ew_v, coef_v,
                 nog0, nig0, nog1, nig1, sem0, sem1):
    cid = lax.axis_index("c")
    sid = lax.axis_index("s")
    wid = cid * NS + sid
    pltpu.sync_copy(src_h.at[pl.ds(wid * CH, CH + 2)], src_v)
    pltpu.sync_copy(dst_h.at[pl.ds(wid * CH, CH + 2)], dst_v)
    pltpu.sync_copy(ew_h.at[pl.ds(wid * CH, CH)], ew_v)

    pltpu.async_copy(no_h.at[src_v.at[0]], nog0, sem0)
    pltpu.async_copy(ni_h.at[dst_v.at[0]], nig0, sem0)
    pltpu.async_copy(no_h.at[src_v.at[1]], nog1, sem1)
    pltpu.async_copy(ni_h.at[dst_v.at[1]], nig1, sem1)

    def half(t, nog, nig, sem):
        pltpu.make_async_copy(no_h.at[src_v.at[t]], nog, sem).wait()
        pltpu.make_async_copy(ni_h.at[dst_v.at[t]], nig, sem).wait()
        for g in range(B // L):
            sl = pl.ds(g * L, L)
            coef_v[t, sl] = nog[sl] * ew_v[t, sl] * nig[sl]
        pltpu.async_copy(no_h.at[src_v.at[t + 2]], nog, sem)
        pltpu.async_copy(ni_h.at[dst_v.at[t + 2]], nig, sem)

    def body(i, c):
        t = 2 * i
        half(t, nog0, nig0, sem0)
        half(t + 1, nog1, nig1, sem1)
        return c
    lax.fori_loop(0, CH // 2, body, 0)

    pltpu.make_async_copy(no_h.at[src_v.at[0]], nog0, sem0).wait()
    pltpu.make_async_copy(ni_h.at[dst_v.at[0]], nig0, sem0).wait()
    pltpu.make_async_copy(no_h.at[src_v.at[1]], nog1, sem1).wait()
    pltpu.make_async_copy(ni_h.at[dst_v.at[1]], nig1, sem1).wait()
    pltpu.sync_copy(coef_v, out_h.at[pl.ds(wid * CH, CH)])


@functools.partial(
    pl.kernel,
    out_type=jax.ShapeDtypeStruct((NC, N_PAD, D), jnp.float32),
    mesh=_mesh(),
    compiler_params=_SC_PARAMS,
    scratch_types=[
        pltpu.VMEM((FB, B), jnp.int32),
        pltpu.VMEM((FB, B), jnp.int32),
        pltpu.VMEM((KB, B, L), jnp.float32),
        [pltpu.VMEM((B, D), jnp.float32)] * KB,
        pltpu.VMEM_SHARED((N_PAD, D), jnp.float32),
        pltpu.SemaphoreType.DMA,
    ],
)
def _prop_kernel(h_h, src_h, dst_h, coefx_h, out_h,
                 src_v, dst_v, coefx_v, bufs, agg_sh, gsem):
    cid = lax.axis_index("c")
    sid = lax.axis_index("s")

    base_b = jnp.where(cid == 0, sid * FB, NS * FB + sid * SB)
    nblk = jnp.where(cid == 0, FB // KB, SB // KB)

    zero_v = bufs[0]

    def zi(i, c):
        zero_v[i // (D // L), pl.ds((i % (D // L)) * L, L)] = (
            jnp.zeros((L,), jnp.float32))
        return c
    lax.fori_loop(0, B * D // L, zi, 0)

    def zc(i, c):
        pltpu.sync_copy(zero_v, agg_sh.at[pl.ds(sid * RPT + i * B, B)])
        return c
    lax.fori_loop(0, RPT // B, zc, 0)

    pltpu.sync_copy(src_h.at[pl.ds(base_b, FB)], src_v)
    pltpu.sync_copy(dst_h.at[pl.ds(base_b, FB)], dst_v)
    plsc.subcore_barrier()

    def blk(bI, c):
        t0 = bI * KB
        for b in range(KB):
            pltpu.async_copy(h_h.at[src_v.at[t0 + b]], bufs[b], gsem)
        pltpu.sync_copy(coefx_h.at[pl.ds(base_b + t0, KB)], coefx_v)
        for b in range(KB):
            pltpu.make_async_copy(
                h_h.at[src_v.at[t0 + b]], bufs[b], gsem).wait()

        for b in range(KB):
            buf = bufs[b]

            def scale(g, ccc):
                for m in range(L):
                    e = g * L + m
                    cvx = coefx_v[b, e]
                    for k in range(D // L):
                        buf[e, pl.ds(k * L, L)] = (
                            buf[e, pl.ds(k * L, L)] * cvx)
                return ccc
            lax.fori_loop(0, B // L, scale, 0)
            pltpu.sync_copy(buf, agg_sh.at[dst_v.at[t0 + b]], add=True)
        return c
    lax.fori_loop(0, nblk, blk, 0)

    plsc.subcore_barrier()
    sl = pl.ds(sid * RPT, RPT)
    pltpu.sync_copy(agg_sh.at[sl], out_h.at[cid, sl])


def _mlp_in_call(x_p, w, b2):
    def body(x_r, w_r, b_r, o_r):
        o_r[...] = jnp.maximum(x_r[...] @ w_r[...] + b_r[...], 0.0)
    return pl.pallas_call(
        body,
        grid=(N_PAD // ROWBLK,),
        in_specs=[
            pl.BlockSpec((ROWBLK, IN_CH), lambda i: (i, 0)),
            pl.BlockSpec((IN_CH, D), lambda i: (0, 0)),
            pl.BlockSpec((1, D), lambda i: (0, 0)),
        ],
        out_specs=pl.BlockSpec((ROWBLK, D), lambda i: (i, 0)),
        out_shape=jax.ShapeDtypeStruct((N_PAD, D), jnp.float32),
    )(x_p, w, b2)


def _expand_call(coef2d, expmat):
    def body(c_r, m_r, o_r):
        o_r[...] = jax.lax.dot(c_r[...], m_r[...],
                               precision=lax.Precision.HIGHEST)
    BLK = 512
    return pl.pallas_call(
        body,
        grid=(TB // BLK,),
        in_specs=[
            pl.BlockSpec((BLK, B), lambda i: (i, 0)),
            pl.BlockSpec((B, B * L), lambda i: (0, 0)),
        ],
        out_specs=pl.BlockSpec((BLK, B * L), lambda i: (i, 0)),
        out_shape=jax.ShapeDtypeStruct((TB, B * L), jnp.float32),
    )(coef2d, expmat)


_DEG_ROWS = 2 * 2 * N_PAD // 128
_NR = N_PAD // 128


def _norm_call(deg_flat):
    def body(d_r, o_r):
        d = d_r[...]
        tot_o = d[0:_NR] + d[2 * _NR:3 * _NR]
        tot_i = d[_NR:2 * _NR] + d[3 * _NR:4 * _NR]
        no = jnp.where(tot_o > 0, lax.rsqrt(tot_o), 0.0)
        ni = jnp.where(tot_i > 0, lax.rsqrt(tot_i), 0.0)
        o_r[...] = jnp.concatenate([no, ni], axis=0)
    return pl.pallas_call(
        body,
        out_shape=jax.ShapeDtypeStruct((2 * _NR, 128), jnp.float32),
    )(deg_flat)


def _combine_call(part, feat0):
    def body(p_r, f_r, o_r):
        p = p_r[...]
        o_r[...] = (1.0 - ALPHA) * (p[0] + p[1]) + ALPHA * f_r[...]
    return pl.pallas_call(
        body,
        grid=(N_PAD // ROWBLK,),
        in_specs=[
            pl.BlockSpec((NC, ROWBLK, D), lambda i: (0, i, 0)),
            pl.BlockSpec((ROWBLK, D), lambda i: (i, 0)),
        ],
        out_specs=pl.BlockSpec((ROWBLK, D), lambda i: (i, 0)),
        out_shape=jax.ShapeDtypeStruct((N_PAD, D), jnp.float32),
    )(part, feat0)


def _final_call(part, feat0, w, b2):
    def body(p_r, f_r, w_r, b_r, o_r):
        p = p_r[...]
        h = (1.0 - ALPHA) * (p[0] + p[1]) + ALPHA * f_r[...]
        o_r[...] = h @ w_r[...] + b_r[...]
    return pl.pallas_call(
        body,
        grid=(N_PAD // ROWBLK,),
        in_specs=[
            pl.BlockSpec((NC, ROWBLK, D), lambda i: (0, i, 0)),
            pl.BlockSpec((ROWBLK, D), lambda i: (i, 0)),
            pl.BlockSpec((D, D), lambda i: (0, 0)),
            pl.BlockSpec((1, D), lambda i: (0, 0)),
        ],
        out_specs=pl.BlockSpec((ROWBLK, D), lambda i: (i, 0)),
        out_shape=jax.ShapeDtypeStruct((N_PAD, D), jnp.float32),
    )(part, feat0, w, b2)


def _extend(a, fill):
    return jnp.concatenate(
        [a, jnp.full((TBX * B - E,), fill, a.dtype)]).reshape(TBX, B)


def kernel(x, edge_index, edge_weight, W_in, b_in, W_out, b_out):
    src_x = _extend(edge_index[0], N)
    dst_x = _extend(edge_index[1], N)
    ew_p = jnp.concatenate(
        [edge_weight, jnp.zeros((E_PAD - E,), jnp.float32)]).reshape(TB, B)
    x_p = jnp.pad(x, ((0, N_PAD - N), (0, 0)))

    deg = _deg_kernel(src_x, dst_x)
    norms = _norm_call(deg.reshape(_DEG_ROWS, 128))
    no = norms[:_NR].reshape(N_PAD)
    ni = norms[_NR:].reshape(N_PAD)

    h0 = _mlp_in_call(x_p, W_in, b_in.reshape(1, D))
    coef = _coef_kernel(src_x, dst_x, ew_p, no, ni)

    expmat = jnp.zeros((B, B * L), jnp.float32).at[
        jnp.arange(B).repeat(L), jnp.arange(B * L)].set(1.0)
    coefx = _expand_call(coef, expmat).reshape(TB, B, L)

    h = h0
    out = None
    for t in range(K):
        part = _prop_kernel(h, src_x, dst_x, coefx)
        if t < K - 1:
            h = _combine_call(part, h0)
        else:
            out = _final_call(part, h0, W_out, b_out.reshape(1, D))
    return out[:N]
